# baseline (speedup 1.0000x reference)
import functools

import jax
import jax.numpy as jnp
from jax import lax
from jax.experimental import pallas as pl
from jax.experimental import pallas as _pl_unused
from jax.experimental.pallas import tpu as pltpu
from jax.experimental.pallas import tpu_sc as plsc

H = 32
A = 1025
B = A * A
T = 3972

K = 16384
NFULL = B // K
KT = B - NFULL * K
KT16 = ((KT + 15) // 16) * 16
BP = NFULL * K + KT16


def _make_sc_gather():
    info = plsc.get_sparse_core_info()
    nc = info.num_cores
    mesh = plsc.VectorSubcoreMesh(core_axis_name="c", subcore_axis_name="s")

    @functools.partial(
        pl.kernel,
        mesh=mesh,
        compiler_params=pltpu.CompilerParams(
            needs_layout_passes=False, use_tc_tiling_on_sc=False),
        out_type=jax.ShapeDtypeStruct((H, B), jnp.float32),
        scratch_types=[
            pltpu.VMEM((T,), jnp.float32),
            pltpu.VMEM((K,), jnp.int32),
            pltpu.VMEM((K,), jnp.float32),
        ],
    )
    def sc_gather(tab_hbm, idx_hbm, out_hbm, col_v, idx_v, out_v):
        wid = lax.axis_index("s") * nc + lax.axis_index("c")
        pltpu.sync_copy(tab_hbm.at[wid], col_v)

        def gather_chunk(n16):
            def body(j, _):
                iv = idx_v[pl.ds(j * 16, 16)]
                out_v[pl.ds(j * 16, 16)] = plsc.load_gather(col_v, [iv])
                return 0
            lax.fori_loop(0, n16, body, 0, unroll=8)

        def chunk(c, _):
            base = c * K
            pltpu.sync_copy(idx_hbm.at[pl.ds(base, K)], idx_v)
            gather_chunk(K // 16)
            pltpu.sync_copy(out_v, out_hbm.at[wid, pl.ds(base, K)])
            return 0

        lax.fori_loop(0, NFULL, chunk, 0)

        tbase = NFULL * K
        pltpu.sync_copy(idx_hbm.at[pl.ds(tbase, KT16)],
                        idx_v.at[pl.ds(0, KT16)])
        gather_chunk(KT16 // 16)
        pltpu.sync_copy(out_v.at[pl.ds(0, KT)],
                        out_hbm.at[wid, pl.ds(tbase, KT)])

    return sc_gather


def kernel(relative_position_bias_table, relative_position_index):
    tab_t = relative_position_bias_table.T.astype(jnp.float32)
    idx = relative_position_index.reshape(-1).astype(jnp.int32)
    idx = jnp.concatenate([idx, jnp.zeros((BP - B,), jnp.int32)])
    out = _make_sc_gather()(tab_t, idx)
    return out.reshape(H, A, A)
